# baseline (speedup 1.0000x reference)
import functools

import jax
import jax.numpy as jnp
from jax.experimental import pallas as pl
from jax.experimental.pallas import tpu as pltpu

N = 10000
NFEAT = 128
NHID = 128
NCLASS = 16

BI = 400
NBLK = N // BI
CACHE_BLKS = 2
OFF = NBLK - CACHE_BLKS

_DOT = functools.partial(
    jax.lax.dot_general,
    dimension_numbers=(((1,), (0,)), ((), ())),
    precision=jax.lax.Precision.DEFAULT,
    preferred_element_type=jnp.float32,
)


def _adj_index(p, i):
    return (jnp.where(p == 0, i, jnp.minimum(i, OFF - 1)), 0)


def _sweep1_body(adj_ref, x_ref, w1_ref, b1_ref, w2_ref, b2_ref,
                 h_ref, s1_scr, s2_scr, cache_scr):
    p = pl.program_id(0)
    i = pl.program_id(1)

    @pl.when((p == 0) & (i == 0))
    def _():
        s1_scr[...] = _DOT(x_ref[...], w1_ref[...]).astype(jnp.bfloat16)

    @pl.when(p == 0)
    def _():
        hh = jnp.maximum(_DOT(adj_ref[...], s1_scr[...]) + b1_ref[...], 0.0)
        s2_scr[pl.ds(i * BI, BI), :] = _DOT(hh, w2_ref[...]).astype(jnp.bfloat16)

    @pl.when((p == 0) & (i >= OFF))
    def _():
        cache_scr[pl.ds((i - OFF) * BI, BI), :] = (
            adj_ref[...].astype(jnp.bfloat16))

    @pl.when((p == 1) & (i < OFF))
    def _():
        h_ref[0] = _DOT(adj_ref[...], s2_scr[...]) + b2_ref[...]

    @pl.when((p == 1) & (i >= OFF))
    def _():
        blk = cache_scr[pl.ds((i - OFF) * BI, BI), :]
        h_ref[0] = _DOT(blk, s2_scr[...]) + b2_ref[...]


def _sweep2_body(adj2_ref, x_ref, w3_ref, b3_ref, w4_ref, b4_ref,
                 h_ref, wla_ref, wlb_ref, bl_ref, out_ref,
                 s3_scr, s4_scr, cache_scr):
    p = pl.program_id(0)
    i = pl.program_id(1)

    @pl.when((p == 0) & (i == 0))
    def _():
        s3_scr[...] = _DOT(x_ref[...], w3_ref[...]).astype(jnp.bfloat16)

    @pl.when(p == 0)
    def _():
        hh = jnp.maximum(_DOT(adj2_ref[...], s3_scr[...]) + b3_ref[...], 0.0)
        s4_scr[pl.ds(i * BI, BI), :] = _DOT(hh, w4_ref[...]).astype(jnp.bfloat16)

    @pl.when((p == 0) & (i >= OFF))
    def _():
        cache_scr[pl.ds((i - OFF) * BI, BI), :] = (
            adj2_ref[...].astype(jnp.bfloat16))

    def _finish(h2raw):
        h2 = h2raw + b4_ref[...]
        h = h_ref[...]
        g = _DOT(h, wla_ref[...]) + _DOT(h2, wlb_ref[...]) + bl_ref[...]
        w = jax.nn.sigmoid(g)
        o = w * h + (1.0 - w) * h2
        m = jnp.max(o, axis=1, keepdims=True)
        e = o - m
        lse = jnp.log(jnp.sum(jnp.exp(e), axis=1, keepdims=True))
        out_ref[0] = e - lse

    @pl.when((p == 1) & (i < OFF))
    def _():
        _finish(_DOT(adj2_ref[...], s4_scr[...]))

    @pl.when((p == 1) & (i >= OFF))
    def _():
        blk = cache_scr[pl.ds((i - OFF) * BI, BI), :]
        _finish(_DOT(blk, s4_scr[...]))


def _rep(shape):
    return pl.BlockSpec(shape, lambda p, i: (0,) * len(shape))


def kernel(x, adj, adj2, W1, b1, W2, b2, W3, b3, W4, b4, Wl, bl):
    f32 = jnp.float32
    bf16 = jnp.bfloat16
    b1r = b1.reshape(1, NHID)
    b3r = b3.reshape(1, NHID)
    b2r = b2.reshape(1, NCLASS)
    b4r = b4.reshape(1, NCLASS)
    blr = bl.reshape(1, NCLASS)
    wla = Wl[:NCLASS]
    wlb = Wl[NCLASS:]
    xb = x.astype(bf16)
    W1b = W1.astype(bf16)
    W3b = W3.astype(bf16)

    grid = (2, NBLK)
    adj_spec = pl.BlockSpec((BI, N), _adj_index)
    blk16 = pl.BlockSpec((BI, NCLASS), lambda p, i: (i, 0))
    out16 = pl.BlockSpec((1, BI, NCLASS), lambda p, i: (p, i, 0))
    params = pltpu.CompilerParams(
        dimension_semantics=("arbitrary", "arbitrary"),
        vmem_limit_bytes=67108864)

    def scratches():
        return [pltpu.VMEM((N, NHID), bf16),
                pltpu.VMEM((N, NCLASS), bf16),
                pltpu.VMEM((CACHE_BLKS * BI, N), bf16)]

    h = pl.pallas_call(
        _sweep1_body,
        grid=grid,
        in_specs=[
            adj_spec,
            _rep((N, NFEAT)),
            _rep((NFEAT, NHID)),
            _rep((1, NHID)),
            _rep((NHID, NCLASS)),
            _rep((1, NCLASS)),
        ],
        out_specs=out16,
        out_shape=jax.ShapeDtypeStruct((2, N, NCLASS), f32),
        scratch_shapes=scratches(),
        compiler_params=params,
    )(adj, xb, W1b, b1r, W2, b2r)
    h = h[1]

    out = pl.pallas_call(
        _sweep2_body,
        grid=grid,
        in_specs=[
            adj_spec,
            _rep((N, NFEAT)),
            _rep((NFEAT, NHID)),
            _rep((1, NHID)),
            _rep((NHID, NCLASS)),
            _rep((1, NCLASS)),
            blk16,
            _rep((NCLASS, NCLASS)),
            _rep((NCLASS, NCLASS)),
            _rep((1, NCLASS)),
        ],
        out_specs=out16,
        out_shape=jax.ShapeDtypeStruct((2, N, NCLASS), f32),
        scratch_shapes=scratches(),
        compiler_params=params,
    )(adj2, xb, W3b, b3r, W4, b4r, h, wla, wlb, blr)

    return out[1]


# speedup vs baseline: 1.0120x; 1.0120x over previous
import functools

import jax
import jax.numpy as jnp
from jax.experimental import pallas as pl
from jax.experimental.pallas import tpu as pltpu

N = 10000
NFEAT = 128
NHID = 128
NCLASS = 16

BI = 400
NBLK = N // BI
CACHE_BLKS = 2
OFF = NBLK - CACHE_BLKS

_DOT = functools.partial(
    jax.lax.dot_general,
    dimension_numbers=(((1,), (0,)), ((), ())),
    precision=jax.lax.Precision.DEFAULT,
    preferred_element_type=jnp.float32,
)


def _adj_index(p, i):
    return (jnp.where(p == 0, i, jnp.minimum(i, OFF - 1)), 0)


def _sweep1_body(adj_ref, x_ref, w1_ref, b1_ref, w2_ref, b2_ref,
                 h_ref, s1_scr, s2_scr, cache_scr):
    p = pl.program_id(0)
    i = pl.program_id(1)

    @pl.when((p == 0) & (i == 0))
    def _():
        s1_scr[...] = _DOT(x_ref[...], w1_ref[...]).astype(jnp.bfloat16)

    @pl.when(p == 0)
    def _():
        hh = jnp.maximum(_DOT(adj_ref[...], s1_scr[...]) + b1_ref[...], 0.0)
        s2_scr[pl.ds(i * BI, BI), :] = _DOT(hh, w2_ref[...]).astype(jnp.bfloat16)

    @pl.when((p == 0) & (i >= OFF))
    def _():
        cache_scr[pl.ds((i - OFF) * BI, BI), :] = (
            adj_ref[...].astype(jnp.bfloat16))

    @pl.when((p == 1) & (i < OFF))
    def _():
        h_ref[0] = _DOT(adj_ref[...], s2_scr[...]) + b2_ref[...]

    @pl.when((p == 1) & (i >= OFF))
    def _():
        blk = cache_scr[pl.ds((i - OFF) * BI, BI), :]
        h_ref[0] = _DOT(blk, s2_scr[...]) + b2_ref[...]


def _sweep2_body(adj2_ref, x_ref, w3_ref, b3_ref, w4_ref, b4_ref,
                 h_ref, wla_ref, wlb_ref, bl_ref, out_ref,
                 s3_scr, s4_scr, cache_scr):
    p = pl.program_id(0)
    i = pl.program_id(1)

    @pl.when((p == 0) & (i == 0))
    def _():
        s3_scr[...] = _DOT(x_ref[...], w3_ref[...]).astype(jnp.bfloat16)

    @pl.when(p == 0)
    def _():
        hh = jnp.maximum(_DOT(adj2_ref[...], s3_scr[...]) + b3_ref[...], 0.0)
        s4_scr[pl.ds(i * BI, BI), :] = _DOT(hh, w4_ref[...]).astype(jnp.bfloat16)

    @pl.when((p == 0) & (i >= OFF))
    def _():
        cache_scr[pl.ds((i - OFF) * BI, BI), :] = (
            adj2_ref[...].astype(jnp.bfloat16))

    def _finish(h2raw):
        h2 = h2raw + b4_ref[...]
        h = h_ref[...]
        g = _DOT(h, wla_ref[...]) + _DOT(h2, wlb_ref[...]) + bl_ref[...]
        w = jax.nn.sigmoid(g)
        o = w * h + (1.0 - w) * h2
        m = jnp.max(o, axis=1, keepdims=True)
        e = o - m
        lse = jnp.log(jnp.sum(jnp.exp(e), axis=1, keepdims=True))
        out_ref[0] = e - lse

    @pl.when((p == 1) & (i < OFF))
    def _():
        _finish(_DOT(adj2_ref[...], s4_scr[...]))

    @pl.when((p == 1) & (i >= OFF))
    def _():
        blk = cache_scr[pl.ds((i - OFF) * BI, BI), :]
        _finish(_DOT(blk, s4_scr[...]))


def _rep(shape):
    return pl.BlockSpec(shape, lambda p, i: (0,) * len(shape))


def kernel(x, adj, adj2, W1, b1, W2, b2, W3, b3, W4, b4, Wl, bl):
    f32 = jnp.float32
    bf16 = jnp.bfloat16
    b1r = b1.reshape(1, NHID)
    b3r = b3.reshape(1, NHID)
    b2r = b2.reshape(1, NCLASS)
    b4r = b4.reshape(1, NCLASS)
    blr = bl.reshape(1, NCLASS)
    wla = Wl[:NCLASS]
    wlb = Wl[NCLASS:]

    grid = (2, NBLK)
    adj_spec = pl.BlockSpec((BI, N), _adj_index)
    blk16 = pl.BlockSpec((BI, NCLASS), lambda p, i: (i, 0))
    out16 = pl.BlockSpec((1, BI, NCLASS), lambda p, i: (p, i, 0))
    params = pltpu.CompilerParams(
        dimension_semantics=("arbitrary", "arbitrary"))

    def scratches():
        return [pltpu.VMEM((N, NHID), bf16),
                pltpu.VMEM((N, NCLASS), bf16),
                pltpu.VMEM((CACHE_BLKS * BI, N), bf16)]

    h = pl.pallas_call(
        _sweep1_body,
        grid=grid,
        in_specs=[
            adj_spec,
            _rep((N, NFEAT)),
            _rep((NFEAT, NHID)),
            _rep((1, NHID)),
            _rep((NHID, NCLASS)),
            _rep((1, NCLASS)),
        ],
        out_specs=out16,
        out_shape=jax.ShapeDtypeStruct((2, N, NCLASS), f32),
        scratch_shapes=scratches(),
        compiler_params=params,
    )(adj, x, W1, b1r, W2, b2r)
    h = h[1]

    out = pl.pallas_call(
        _sweep2_body,
        grid=grid,
        in_specs=[
            adj_spec,
            _rep((N, NFEAT)),
            _rep((NFEAT, NHID)),
            _rep((1, NHID)),
            _rep((NHID, NCLASS)),
            _rep((1, NCLASS)),
            blk16,
            _rep((NCLASS, NCLASS)),
            _rep((NCLASS, NCLASS)),
            _rep((1, NCLASS)),
        ],
        out_specs=out16,
        out_shape=jax.ShapeDtypeStruct((2, N, NCLASS), f32),
        scratch_shapes=scratches(),
        compiler_params=params,
    )(adj2, x, W3, b3r, W4, b4r, h, wla, wlb, blr)

    return out[1]
